# initial kernel scaffold (unmeasured)
import jax
import jax.numpy as jnp
from jax import lax
from jax.experimental import pallas as pl
from jax.experimental.pallas import tpu as pltpu

N_DEV = 16


def _silu(y):
    return y * jax.nn.sigmoid(y)


def kernel(x, w_mat):
    m_per, k = x.shape
    _, n_per = w_mat.shape

    def body(x_ref, w_ref, out_ref, comm_ref, cbuf_ref,
             send_sem, recv_sems, local_sems):
        my = lax.axis_index("i")
        left = lax.rem(my + N_DEV - 1, N_DEV)
        right = lax.rem(my + 1, N_DEV)

        barrier_sem = pltpu.get_barrier_semaphore()
        for nbr in (left, right):
            pl.semaphore_signal(
                barrier_sem, inc=1,
                device_id=(nbr,), device_id_type=pl.DeviceIdType.MESH,
            )
        pl.semaphore_wait(barrier_sem, 2)

        pending = pltpu.make_async_remote_copy(
            src_ref=x_ref,
            dst_ref=comm_ref.at[my],
            send_sem=send_sem,
            recv_sem=recv_sems.at[0],
            device_id=(right,),
            device_id_type=pl.DeviceIdType.MESH,
        )
        pending.start()

        out_ref[pl.ds(my * m_per, m_per), :] = _silu(
            jnp.dot(x_ref[...], w_ref[...], preferred_element_type=jnp.float32)
        )

        for h in range(N_DEV - 1):
            origin = lax.rem(my - 1 - h + 2 * N_DEV, N_DEV)
            recv = pltpu.make_async_remote_copy(
                src_ref=comm_ref.at[origin],
                dst_ref=comm_ref.at[origin],
                send_sem=send_sem,
                recv_sem=recv_sems.at[h],
                device_id=(left,),
                device_id_type=pl.DeviceIdType.MESH,
            )
            recv.wait_recv()

            if h < N_DEV - 2:
                pending.wait_send()
                pending = pltpu.make_async_remote_copy(
                    src_ref=comm_ref.at[origin],
                    dst_ref=comm_ref.at[origin],
                    send_sem=send_sem,
                    recv_sem=recv_sems.at[h + 1],
                    device_id=(right,),
                    device_id_type=pl.DeviceIdType.MESH,
                )
                pending.start()

            cp = pltpu.make_async_copy(
                comm_ref.at[origin], cbuf_ref.at[h % 2], local_sems.at[h % 2]
            )
            cp.start()
            cp.wait()
            out_ref[pl.ds(origin * m_per, m_per), :] = _silu(
                jnp.dot(cbuf_ref[h % 2], w_ref[...],
                        preferred_element_type=jnp.float32)
            )

        pending.wait_send()

    out_shape = jax.ShapeDtypeStruct((N_DEV * m_per, n_per), jnp.float32)
    return pl.pallas_call(
        body,
        out_shape=out_shape,
        in_specs=[
            pl.BlockSpec(memory_space=pltpu.VMEM),
            pl.BlockSpec(memory_space=pltpu.VMEM),
        ],
        out_specs=pl.BlockSpec(memory_space=pltpu.VMEM),
        scratch_shapes=[
            pltpu.HBM((N_DEV, m_per, k), jnp.float32),
            pltpu.VMEM((2, m_per, k), jnp.float32),
            pltpu.SemaphoreType.DMA,
            pltpu.SemaphoreType.DMA((N_DEV - 1,)),
            pltpu.SemaphoreType.DMA((2,)),
        ],
        compiler_params=pltpu.CompilerParams(collective_id=0),
    )(x, w_mat)


# baseline (device time: 728670 ns/iter reference)
import jax
import jax.numpy as jnp
from jax import lax
from jax.experimental import pallas as pl
from jax.experimental.pallas import tpu as pltpu

N_DEV = 16


def _silu(y):
    return y * jax.nn.sigmoid(y)


def kernel(x, w_mat):
    m_per, k = x.shape
    _, n_per = w_mat.shape

    def body(x_ref, w_ref, out_ref, comm_ref, cbuf_ref,
             send_sem, recv_sems, local_sems):
        my = lax.axis_index("i")
        left = lax.rem(my + N_DEV - 1, N_DEV)
        right = lax.rem(my + 1, N_DEV)

        barrier_sem = pltpu.get_barrier_semaphore()
        for nbr in (left, right):
            pl.semaphore_signal(
                barrier_sem, inc=1,
                device_id=(nbr,), device_id_type=pl.DeviceIdType.MESH,
            )
        pl.semaphore_wait(barrier_sem, 2)

        pending = pltpu.make_async_remote_copy(
            src_ref=x_ref,
            dst_ref=comm_ref.at[my],
            send_sem=send_sem,
            recv_sem=recv_sems.at[0],
            device_id=(right,),
            device_id_type=pl.DeviceIdType.MESH,
        )
        pending.start()

        out_ref[pl.ds(my * m_per, m_per), :] = _silu(
            jnp.dot(x_ref[...], w_ref[...], preferred_element_type=jnp.float32)
        )

        for h in range(N_DEV - 1):
            origin = lax.rem(my - 1 - h + 2 * N_DEV, N_DEV)
            recv = pltpu.make_async_remote_copy(
                src_ref=comm_ref.at[origin],
                dst_ref=comm_ref.at[origin],
                send_sem=send_sem,
                recv_sem=recv_sems.at[h],
                device_id=(left,),
                device_id_type=pl.DeviceIdType.MESH,
            )
            recv.wait_recv()

            if h < N_DEV - 2:
                pending.wait_send()
                pending = pltpu.make_async_remote_copy(
                    src_ref=comm_ref.at[origin],
                    dst_ref=comm_ref.at[origin],
                    send_sem=send_sem,
                    recv_sem=recv_sems.at[h + 1],
                    device_id=(right,),
                    device_id_type=pl.DeviceIdType.MESH,
                )
                pending.start()

            cp = pltpu.make_async_copy(
                comm_ref.at[origin], cbuf_ref.at[h % 2], local_sems.at[h % 2]
            )
            cp.start()
            cp.wait()
            out_ref[pl.ds(origin * m_per, m_per), :] = _silu(
                jnp.dot(cbuf_ref[h % 2], w_ref[...],
                        preferred_element_type=jnp.float32)
            )

        pending.wait_send()

    out_shapes = [
        jax.ShapeDtypeStruct((N_DEV * m_per, n_per), jnp.float32),
        jax.ShapeDtypeStruct((N_DEV, m_per, k), jnp.float32),
    ]
    out, _ = pl.pallas_call(
        body,
        out_shape=out_shapes,
        in_specs=[
            pl.BlockSpec(memory_space=pltpu.VMEM),
            pl.BlockSpec(memory_space=pltpu.VMEM),
        ],
        out_specs=[
            pl.BlockSpec(memory_space=pltpu.VMEM),
            pl.BlockSpec(memory_space=pltpu.HBM),
        ],
        scratch_shapes=[
            pltpu.VMEM((2, m_per, k), jnp.float32),
            pltpu.SemaphoreType.DMA,
            pltpu.SemaphoreType.DMA((N_DEV - 1,)),
            pltpu.SemaphoreType.DMA((2,)),
        ],
        compiler_params=pltpu.CompilerParams(collective_id=0),
    )(x, w_mat)
    return out


# device time: 397163 ns/iter; 1.8347x vs baseline; 1.8347x over previous
import jax
import jax.numpy as jnp
from jax import lax
from jax.experimental import pallas as pl
from jax.experimental.pallas import tpu as pltpu

N_DEV = 16
H_R = 8
H_L = 7


def _silu(y):
    return y * jax.nn.sigmoid(y)


def kernel(x, w_mat):
    m_per, k = x.shape
    _, n_per = w_mat.shape

    def body(x_ref, w_ref, out_ref, comm_ref, cbuf_ref,
             send_sem_r, send_sem_l, recv_sems_r, recv_sems_l, local_sems):
        my = lax.axis_index("i")
        left = lax.rem(my + N_DEV - 1, N_DEV)
        right = lax.rem(my + 1, N_DEV)

        def gemm(origin, src_ref):
            out_ref[pl.ds(origin * m_per, m_per), :] = _silu(
                jnp.dot(src_ref[...], w_ref[...],
                        preferred_element_type=jnp.float32)
            )

        barrier_sem = pltpu.get_barrier_semaphore()
        for nbr in (left, right):
            pl.semaphore_signal(
                barrier_sem, inc=1,
                device_id=(nbr,), device_id_type=pl.DeviceIdType.MESH,
            )
        pl.semaphore_wait(barrier_sem, 2)

        pend_r = pltpu.make_async_remote_copy(
            src_ref=x_ref, dst_ref=comm_ref.at[my],
            send_sem=send_sem_r, recv_sem=recv_sems_r.at[0],
            device_id=(right,), device_id_type=pl.DeviceIdType.MESH,
        )
        pend_r.start()
        pend_l = pltpu.make_async_remote_copy(
            src_ref=x_ref, dst_ref=comm_ref.at[my],
            send_sem=send_sem_l, recv_sem=recv_sems_l.at[0],
            device_id=(left,), device_id_type=pl.DeviceIdType.MESH,
        )
        pend_l.start()

        gemm(my, x_ref)

        for h in range(H_R):
            o_r = lax.rem(my - 1 - h + 2 * N_DEV, N_DEV)
            recv_r = pltpu.make_async_remote_copy(
                src_ref=comm_ref.at[o_r], dst_ref=comm_ref.at[o_r],
                send_sem=send_sem_r, recv_sem=recv_sems_r.at[h],
                device_id=(left,), device_id_type=pl.DeviceIdType.MESH,
            )
            recv_r.wait_recv()
            if h < H_R - 1:
                pend_r.wait_send()
                pend_r = pltpu.make_async_remote_copy(
                    src_ref=comm_ref.at[o_r], dst_ref=comm_ref.at[o_r],
                    send_sem=send_sem_r, recv_sem=recv_sems_r.at[h + 1],
                    device_id=(right,), device_id_type=pl.DeviceIdType.MESH,
                )
                pend_r.start()
            cp_r = pltpu.make_async_copy(
                comm_ref.at[o_r], cbuf_ref.at[h % 2], local_sems.at[h % 2]
            )
            cp_r.start()

            if h < H_L:
                o_l = lax.rem(my + 1 + h, N_DEV)
                recv_l = pltpu.make_async_remote_copy(
                    src_ref=comm_ref.at[o_l], dst_ref=comm_ref.at[o_l],
                    send_sem=send_sem_l, recv_sem=recv_sems_l.at[h],
                    device_id=(right,), device_id_type=pl.DeviceIdType.MESH,
                )
                recv_l.wait_recv()
                if h < H_L - 1:
                    pend_l.wait_send()
                    pend_l = pltpu.make_async_remote_copy(
                        src_ref=comm_ref.at[o_l], dst_ref=comm_ref.at[o_l],
                        send_sem=send_sem_l, recv_sem=recv_sems_l.at[h + 1],
                        device_id=(left,), device_id_type=pl.DeviceIdType.MESH,
                    )
                    pend_l.start()
                cp_l = pltpu.make_async_copy(
                    comm_ref.at[o_l], cbuf_ref.at[2 + h % 2],
                    local_sems.at[2 + h % 2]
                )
                cp_l.start()

            cp_r.wait()
            gemm(o_r, cbuf_ref.at[h % 2])
            if h < H_L:
                cp_l.wait()
                gemm(o_l, cbuf_ref.at[2 + h % 2])

        pend_r.wait_send()
        pend_l.wait_send()

    out_shapes = [
        jax.ShapeDtypeStruct((N_DEV * m_per, n_per), jnp.float32),
        jax.ShapeDtypeStruct((N_DEV, m_per, k), jnp.float32),
    ]
    out, _ = pl.pallas_call(
        body,
        out_shape=out_shapes,
        in_specs=[
            pl.BlockSpec(memory_space=pltpu.VMEM),
            pl.BlockSpec(memory_space=pltpu.VMEM),
        ],
        out_specs=[
            pl.BlockSpec(memory_space=pltpu.VMEM),
            pl.BlockSpec(memory_space=pltpu.HBM),
        ],
        scratch_shapes=[
            pltpu.VMEM((4, m_per, k), jnp.float32),
            pltpu.SemaphoreType.DMA,
            pltpu.SemaphoreType.DMA,
            pltpu.SemaphoreType.DMA((H_R,)),
            pltpu.SemaphoreType.DMA((H_L,)),
            pltpu.SemaphoreType.DMA((4,)),
        ],
        compiler_params=pltpu.CompilerParams(collective_id=0),
    )(x, w_mat)
    return out


# device time: 376274 ns/iter; 1.9365x vs baseline; 1.0555x over previous
import jax
import jax.numpy as jnp
from jax import lax
from jax.experimental import pallas as pl
from jax.experimental.pallas import tpu as pltpu

N_DEV = 16
H_FULL = 7


def _silu(y):
    return y * jax.nn.sigmoid(y)


def kernel(x, w_mat):
    m_per, k = x.shape
    _, n_per = w_mat.shape
    m_half = m_per // 2

    def body(x_ref, w_ref, out_ref, comm_ref, cbuf_ref,
             send_sem_r, send_sem_l, recv_sems_r, recv_sems_l, local_sems):
        my = lax.axis_index("i")
        left = lax.rem(my + N_DEV - 1, N_DEV)
        right = lax.rem(my + 1, N_DEV)

        def gemm(origin, src_ref):
            out_ref[pl.ds(origin * m_per, m_per), :] = _silu(
                jnp.dot(src_ref[...], w_ref[...],
                        preferred_element_type=jnp.float32)
            )

        barrier_sem = pltpu.get_barrier_semaphore()
        for nbr in (left, right):
            pl.semaphore_signal(
                barrier_sem, inc=1,
                device_id=(nbr,), device_id_type=pl.DeviceIdType.MESH,
            )
        pl.semaphore_wait(barrier_sem, 2)

        pend_r = pltpu.make_async_remote_copy(
            src_ref=x_ref, dst_ref=comm_ref.at[my],
            send_sem=send_sem_r, recv_sem=recv_sems_r.at[0],
            device_id=(right,), device_id_type=pl.DeviceIdType.MESH,
        )
        pend_r.start()
        pend_l = pltpu.make_async_remote_copy(
            src_ref=x_ref, dst_ref=comm_ref.at[my],
            send_sem=send_sem_l, recv_sem=recv_sems_l.at[0],
            device_id=(left,), device_id_type=pl.DeviceIdType.MESH,
        )
        pend_l.start()

        gemm(my, x_ref)

        for h in range(H_FULL):
            o_r = lax.rem(my - 1 - h + 2 * N_DEV, N_DEV)
            recv_r = pltpu.make_async_remote_copy(
                src_ref=comm_ref.at[o_r], dst_ref=comm_ref.at[o_r],
                send_sem=send_sem_r, recv_sem=recv_sems_r.at[h],
                device_id=(left,), device_id_type=pl.DeviceIdType.MESH,
            )
            recv_r.wait_recv()
            pend_r.wait_send()
            if h < H_FULL - 1:
                pend_r = pltpu.make_async_remote_copy(
                    src_ref=comm_ref.at[o_r], dst_ref=comm_ref.at[o_r],
                    send_sem=send_sem_r, recv_sem=recv_sems_r.at[h + 1],
                    device_id=(right,), device_id_type=pl.DeviceIdType.MESH,
                )
            else:
                pend_r = pltpu.make_async_remote_copy(
                    src_ref=comm_ref.at[o_r, pl.ds(0, m_half)],
                    dst_ref=comm_ref.at[o_r, pl.ds(0, m_half)],
                    send_sem=send_sem_r, recv_sem=recv_sems_r.at[h + 1],
                    device_id=(right,), device_id_type=pl.DeviceIdType.MESH,
                )
            pend_r.start()
            cp_r = pltpu.make_async_copy(
                comm_ref.at[o_r], cbuf_ref.at[h % 2], local_sems.at[h % 2]
            )
            cp_r.start()

            o_l = lax.rem(my + 1 + h, N_DEV)
            recv_l = pltpu.make_async_remote_copy(
                src_ref=comm_ref.at[o_l], dst_ref=comm_ref.at[o_l],
                send_sem=send_sem_l, recv_sem=recv_sems_l.at[h],
                device_id=(right,), device_id_type=pl.DeviceIdType.MESH,
            )
            recv_l.wait_recv()
            pend_l.wait_send()
            if h < H_FULL - 1:
                pend_l = pltpu.make_async_remote_copy(
                    src_ref=comm_ref.at[o_l], dst_ref=comm_ref.at[o_l],
                    send_sem=send_sem_l, recv_sem=recv_sems_l.at[h + 1],
                    device_id=(left,), device_id_type=pl.DeviceIdType.MESH,
                )
            else:
                pend_l = pltpu.make_async_remote_copy(
                    src_ref=comm_ref.at[o_l, pl.ds(m_half, m_half)],
                    dst_ref=comm_ref.at[o_l, pl.ds(m_half, m_half)],
                    send_sem=send_sem_l, recv_sem=recv_sems_l.at[h + 1],
                    device_id=(left,), device_id_type=pl.DeviceIdType.MESH,
                )
            pend_l.start()
            cp_l = pltpu.make_async_copy(
                comm_ref.at[o_l], cbuf_ref.at[2 + h % 2],
                local_sems.at[2 + h % 2]
            )
            cp_l.start()

            cp_r.wait()
            gemm(o_r, cbuf_ref.at[h % 2])
            cp_l.wait()
            gemm(o_l, cbuf_ref.at[2 + h % 2])

        o_a = lax.rem(my + N_DEV // 2, N_DEV)
        recv_a_r = pltpu.make_async_remote_copy(
            src_ref=comm_ref.at[o_a, pl.ds(0, m_half)],
            dst_ref=comm_ref.at[o_a, pl.ds(0, m_half)],
            send_sem=send_sem_r, recv_sem=recv_sems_r.at[H_FULL],
            device_id=(left,), device_id_type=pl.DeviceIdType.MESH,
        )
        recv_a_r.wait_recv()
        recv_a_l = pltpu.make_async_remote_copy(
            src_ref=comm_ref.at[o_a, pl.ds(m_half, m_half)],
            dst_ref=comm_ref.at[o_a, pl.ds(m_half, m_half)],
            send_sem=send_sem_l, recv_sem=recv_sems_l.at[H_FULL],
            device_id=(right,), device_id_type=pl.DeviceIdType.MESH,
        )
        recv_a_l.wait_recv()
        cp_a = pltpu.make_async_copy(
            comm_ref.at[o_a], cbuf_ref.at[0], local_sems.at[0]
        )
        cp_a.start()
        cp_a.wait()
        gemm(o_a, cbuf_ref.at[0])

        pend_r.wait_send()
        pend_l.wait_send()

    out_shapes = [
        jax.ShapeDtypeStruct((N_DEV * m_per, n_per), jnp.float32),
        jax.ShapeDtypeStruct((N_DEV, m_per, k), jnp.float32),
    ]
    out, _ = pl.pallas_call(
        body,
        out_shape=out_shapes,
        in_specs=[
            pl.BlockSpec(memory_space=pltpu.VMEM),
            pl.BlockSpec(memory_space=pltpu.VMEM),
        ],
        out_specs=[
            pl.BlockSpec(memory_space=pltpu.VMEM),
            pl.BlockSpec(memory_space=pltpu.HBM),
        ],
        scratch_shapes=[
            pltpu.VMEM((4, m_per, k), jnp.float32),
            pltpu.SemaphoreType.DMA,
            pltpu.SemaphoreType.DMA,
            pltpu.SemaphoreType.DMA((H_FULL + 1,)),
            pltpu.SemaphoreType.DMA((H_FULL + 1,)),
            pltpu.SemaphoreType.DMA((4,)),
        ],
        compiler_params=pltpu.CompilerParams(collective_id=0),
    )(x, w_mat)
    return out


# device time: 375311 ns/iter; 1.9415x vs baseline; 1.0026x over previous
import jax
import jax.numpy as jnp
from jax import lax
from jax.experimental import pallas as pl
from jax.experimental.pallas import tpu as pltpu

N_DEV = 16
H_FULL = 7


def _silu(y):
    return y * jax.nn.sigmoid(y)


def kernel(x, w_mat):
    m_per, k = x.shape
    _, n_per = w_mat.shape
    m_half = m_per // 2

    def body(x_ref, w_ref, out_ref, comm_ref, cbuf_ref, wbf_ref,
             send_sem_r, send_sem_l, recv_sems_r, recv_sems_l, local_sems):
        my = lax.axis_index("i")
        left = lax.rem(my + N_DEV - 1, N_DEV)
        right = lax.rem(my + 1, N_DEV)

        wbf_ref[...] = w_ref[...].astype(jnp.bfloat16)

        def gemm(row_start, src):
            out_ref[pl.ds(row_start, src.shape[0]), :] = _silu(
                jnp.dot(src.astype(jnp.bfloat16), wbf_ref[...],
                        preferred_element_type=jnp.float32)
            )

        barrier_sem = pltpu.get_barrier_semaphore()
        for nbr in (left, right):
            pl.semaphore_signal(
                barrier_sem, inc=1,
                device_id=(nbr,), device_id_type=pl.DeviceIdType.MESH,
            )
        pl.semaphore_wait(barrier_sem, 2)

        pend_r = pltpu.make_async_remote_copy(
            src_ref=x_ref, dst_ref=comm_ref.at[my],
            send_sem=send_sem_r, recv_sem=recv_sems_r.at[0],
            device_id=(right,), device_id_type=pl.DeviceIdType.MESH,
        )
        pend_r.start()
        pend_l = pltpu.make_async_remote_copy(
            src_ref=x_ref, dst_ref=comm_ref.at[my],
            send_sem=send_sem_l, recv_sem=recv_sems_l.at[0],
            device_id=(left,), device_id_type=pl.DeviceIdType.MESH,
        )
        pend_l.start()

        gemm(my * m_per, x_ref[...])

        for h in range(H_FULL):
            o_r = lax.rem(my - 1 - h + 2 * N_DEV, N_DEV)
            recv_r = pltpu.make_async_remote_copy(
                src_ref=comm_ref.at[o_r], dst_ref=comm_ref.at[o_r],
                send_sem=send_sem_r, recv_sem=recv_sems_r.at[h],
                device_id=(left,), device_id_type=pl.DeviceIdType.MESH,
            )
            recv_r.wait_recv()
            pend_r.wait_send()
            if h < H_FULL - 1:
                pend_r = pltpu.make_async_remote_copy(
                    src_ref=comm_ref.at[o_r], dst_ref=comm_ref.at[o_r],
                    send_sem=send_sem_r, recv_sem=recv_sems_r.at[h + 1],
                    device_id=(right,), device_id_type=pl.DeviceIdType.MESH,
                )
            else:
                pend_r = pltpu.make_async_remote_copy(
                    src_ref=comm_ref.at[o_r, pl.ds(0, m_half)],
                    dst_ref=comm_ref.at[o_r, pl.ds(0, m_half)],
                    send_sem=send_sem_r, recv_sem=recv_sems_r.at[h + 1],
                    device_id=(right,), device_id_type=pl.DeviceIdType.MESH,
                )
            pend_r.start()
            cp_r = pltpu.make_async_copy(
                comm_ref.at[o_r], cbuf_ref.at[h % 2], local_sems.at[h % 2]
            )
            cp_r.start()

            o_l = lax.rem(my + 1 + h, N_DEV)
            recv_l = pltpu.make_async_remote_copy(
                src_ref=comm_ref.at[o_l], dst_ref=comm_ref.at[o_l],
                send_sem=send_sem_l, recv_sem=recv_sems_l.at[h],
                device_id=(right,), device_id_type=pl.DeviceIdType.MESH,
            )
            recv_l.wait_recv()
            pend_l.wait_send()
            if h < H_FULL - 1:
                pend_l = pltpu.make_async_remote_copy(
                    src_ref=comm_ref.at[o_l], dst_ref=comm_ref.at[o_l],
                    send_sem=send_sem_l, recv_sem=recv_sems_l.at[h + 1],
                    device_id=(left,), device_id_type=pl.DeviceIdType.MESH,
                )
            else:
                pend_l = pltpu.make_async_remote_copy(
                    src_ref=comm_ref.at[o_l, pl.ds(m_half, m_half)],
                    dst_ref=comm_ref.at[o_l, pl.ds(m_half, m_half)],
                    send_sem=send_sem_l, recv_sem=recv_sems_l.at[h + 1],
                    device_id=(left,), device_id_type=pl.DeviceIdType.MESH,
                )
            pend_l.start()
            cp_l = pltpu.make_async_copy(
                comm_ref.at[o_l], cbuf_ref.at[2 + h % 2],
                local_sems.at[2 + h % 2]
            )
            cp_l.start()

            cp_r.wait()
            gemm(o_r * m_per, cbuf_ref[h % 2])
            cp_l.wait()
            gemm(o_l * m_per, cbuf_ref[2 + h % 2])

        o_a = lax.rem(my + N_DEV // 2, N_DEV)
        recv_a_r = pltpu.make_async_remote_copy(
            src_ref=comm_ref.at[o_a, pl.ds(0, m_half)],
            dst_ref=comm_ref.at[o_a, pl.ds(0, m_half)],
            send_sem=send_sem_r, recv_sem=recv_sems_r.at[H_FULL],
            device_id=(left,), device_id_type=pl.DeviceIdType.MESH,
        )
        recv_a_r.wait_recv()
        cp_a_r = pltpu.make_async_copy(
            comm_ref.at[o_a, pl.ds(0, m_half)],
            cbuf_ref.at[0, pl.ds(0, m_half)], local_sems.at[0]
        )
        cp_a_r.start()
        recv_a_l = pltpu.make_async_remote_copy(
            src_ref=comm_ref.at[o_a, pl.ds(m_half, m_half)],
            dst_ref=comm_ref.at[o_a, pl.ds(m_half, m_half)],
            send_sem=send_sem_l, recv_sem=recv_sems_l.at[H_FULL],
            device_id=(right,), device_id_type=pl.DeviceIdType.MESH,
        )
        cp_a_r.wait()
        gemm(o_a * m_per, cbuf_ref[0, :m_half])
        recv_a_l.wait_recv()
        cp_a_l = pltpu.make_async_copy(
            comm_ref.at[o_a, pl.ds(m_half, m_half)],
            cbuf_ref.at[1, pl.ds(0, m_half)], local_sems.at[1]
        )
        cp_a_l.start()
        cp_a_l.wait()
        gemm(o_a * m_per + m_half, cbuf_ref[1, :m_half])

        pend_r.wait_send()
        pend_l.wait_send()

    out_shapes = [
        jax.ShapeDtypeStruct((N_DEV * m_per, n_per), jnp.float32),
        jax.ShapeDtypeStruct((N_DEV, m_per, k), jnp.float32),
    ]
    out, _ = pl.pallas_call(
        body,
        out_shape=out_shapes,
        in_specs=[
            pl.BlockSpec(memory_space=pltpu.VMEM),
            pl.BlockSpec(memory_space=pltpu.VMEM),
        ],
        out_specs=[
            pl.BlockSpec(memory_space=pltpu.VMEM),
            pl.BlockSpec(memory_space=pltpu.HBM),
        ],
        scratch_shapes=[
            pltpu.VMEM((4, m_per, k), jnp.float32),
            pltpu.VMEM((k, n_per), jnp.bfloat16),
            pltpu.SemaphoreType.DMA,
            pltpu.SemaphoreType.DMA,
            pltpu.SemaphoreType.DMA((H_FULL + 1,)),
            pltpu.SemaphoreType.DMA((H_FULL + 1,)),
            pltpu.SemaphoreType.DMA((4,)),
        ],
        compiler_params=pltpu.CompilerParams(collective_id=0),
    )(x, w_mat)
    return out
